# Initial kernel scaffold; baseline (speedup 1.0000x reference)
import jax
import jax.numpy as jnp
from jax.experimental import pallas as pl


def kernel(data):
    raise NotImplementedError("write your pallas kernel here")


# speedup vs baseline: 27.3456x; 27.3456x over previous
import jax
import jax.numpy as jnp
from jax import lax
from jax.experimental import pallas as pl
from jax.experimental.pallas import tpu as pltpu

_ROWS = 64
_COLS = 32768
_N = _ROWS * _COLS
_K = max(50, _N // 100)


def _select_sum_body(x_ref, out_ref, key_ref):
    x = x_ref[...]
    s = lax.bitcast_convert_type(x, jnp.int32)
    key = jnp.where(s < 0, s ^ jnp.int32(0x7FFFFFFF), s)
    key_ref[...] = key

    def bit_step(i, pkey):
        b = 31 - i
        qkey = pkey + (jnp.int32(1) << b)
        c = jnp.sum((key_ref[...] >= qkey).astype(jnp.int32))
        return jnp.where(c >= _K, qkey, pkey)

    pkey = lax.fori_loop(0, 32, bit_step, jnp.int32(-(2**31)))

    t_bits = jnp.where(pkey < 0, pkey ^ jnp.int32(0x7FFFFFFF), pkey)
    t = lax.bitcast_convert_type(t_bits, jnp.float32)
    keys = key_ref[...]
    above = keys > pkey
    c_above = jnp.sum(above.astype(jnp.int32))
    s_above = jnp.sum(jnp.where(above, x_ref[...], jnp.float32(0.0)))
    out_ref[0, 0] = s_above + (jnp.int32(_K) - c_above).astype(jnp.float32) * t


def kernel(data):
    out = pl.pallas_call(
        _select_sum_body,
        out_shape=jax.ShapeDtypeStruct((1, 1), jnp.float32),
        in_specs=[pl.BlockSpec(memory_space=pltpu.VMEM)],
        out_specs=pl.BlockSpec(memory_space=pltpu.SMEM),
        scratch_shapes=[pltpu.VMEM((_ROWS, _COLS), jnp.int32)],
    )(data)
    return out[0, 0]
